# baseline (speedup 1.0000x reference)
import functools

import jax
import jax.numpy as jnp
from jax import lax
from jax.experimental import pallas as pl
from jax.experimental.pallas import tpu as pltpu
from jax.experimental.pallas import tpu_sc as plsc

VOCAB = 1000000
LATENT = 32
BATCH = 16384

NC = 2
NS = 16
NW = NC * NS

ROWS = BATCH // NW
KCH = 128
NCH = ROWS // KCH


def _sc_lookup(id1, id2, id3, table):
    mesh = plsc.VectorSubcoreMesh(core_axis_name="c", subcore_axis_name="s")

    @functools.partial(
        pl.kernel,
        mesh=mesh,
        compiler_params=pltpu.CompilerParams(use_tc_tiling_on_sc=False),
        out_type=(
            jax.ShapeDtypeStruct((BATCH, LATENT), jnp.float32),
            jax.ShapeDtypeStruct((BATCH, LATENT), jnp.float32),
            jax.ShapeDtypeStruct((BATCH, LATENT), jnp.float32),
        ),
        scratch_types=[
            pltpu.VMEM((NCH, KCH), jnp.int32),
            pltpu.VMEM((NCH, KCH), jnp.int32),
            pltpu.VMEM((NCH, KCH), jnp.int32),
            pltpu.VMEM((ROWS, LATENT), jnp.float32),
            pltpu.VMEM((ROWS, LATENT), jnp.float32),
            pltpu.VMEM((ROWS, LATENT), jnp.float32),
            pltpu.SemaphoreType.DMA,
        ],
    )
    def k(id1_hbm, id2_hbm, id3_hbm, table_hbm, out1, out2, out3,
          idx1_v, idx2_v, idx3_v, rows1_v, rows2_v, rows3_v, sem):
        wid = lax.axis_index("s") * NC + lax.axis_index("c")
        base = wid * ROWS

        for j in range(NCH):
            pltpu.sync_copy(id1_hbm.at[pl.ds(base + j * KCH, KCH)], idx1_v.at[j])
            pltpu.sync_copy(id2_hbm.at[pl.ds(base + j * KCH, KCH)], idx2_v.at[j])
            pltpu.sync_copy(id3_hbm.at[pl.ds(base + j * KCH, KCH)], idx3_v.at[j])

        copies = []
        for j in range(NCH):
            copies.append(pltpu.make_async_copy(
                table_hbm.at[idx1_v.at[j]],
                rows1_v.at[pl.ds(j * KCH, KCH)], sem))
            copies.append(pltpu.make_async_copy(
                table_hbm.at[idx2_v.at[j]],
                rows2_v.at[pl.ds(j * KCH, KCH)], sem))
            copies.append(pltpu.make_async_copy(
                table_hbm.at[idx3_v.at[j]],
                rows3_v.at[pl.ds(j * KCH, KCH)], sem))
        for cp in copies:
            cp.start()
        for cp in copies:
            cp.wait()

        pltpu.sync_copy(rows1_v, out1.at[pl.ds(base, ROWS)])
        pltpu.sync_copy(rows2_v, out2.at[pl.ds(base, ROWS)])
        pltpu.sync_copy(rows3_v, out3.at[pl.ds(base, ROWS)])

    return k(id1, id2, id3, table)


def kernel(input_ids, table):
    ids = input_ids.astype(jnp.int32)
    id1 = ids[:, 0]
    id2 = ids[:, 1]
    id3 = jnp.concatenate([id2[-1:], id2[:-1]])
    return _sc_lookup(id1, id2, id3, table)


# speedup vs baseline: 1.0017x; 1.0017x over previous
import functools

import jax
import jax.numpy as jnp
from jax import lax
from jax.experimental import pallas as pl
from jax.experimental.pallas import tpu as pltpu
from jax.experimental.pallas import tpu_sc as plsc

VOCAB = 1000000
LATENT = 32
BATCH = 16384

NC = 2
NS = 16
NW = NC * NS
L = 16

ROWS = BATCH // NW
KCH = 128
NCH = ROWS // KCH
PAD = 8


def _sc_lookup(ids_flat, table):
    mesh = plsc.VectorSubcoreMesh(core_axis_name="c", subcore_axis_name="s")

    @functools.partial(
        pl.kernel,
        mesh=mesh,
        compiler_params=pltpu.CompilerParams(
            use_tc_tiling_on_sc=False,
            needs_layout_passes=False,
        ),
        out_type=(
            jax.ShapeDtypeStruct((BATCH, LATENT), jnp.float32),
            jax.ShapeDtypeStruct((BATCH, LATENT), jnp.float32),
            jax.ShapeDtypeStruct((BATCH, LATENT), jnp.float32),
        ),
        scratch_types=[
            pltpu.VMEM((2 * ROWS + PAD,), jnp.int32),
            pltpu.VMEM((NCH, KCH), jnp.int32),
            pltpu.VMEM((NCH, KCH), jnp.int32),
            pltpu.VMEM((NCH, KCH), jnp.int32),
            pltpu.VMEM((ROWS, LATENT), jnp.float32),
            pltpu.VMEM((ROWS, LATENT), jnp.float32),
            pltpu.VMEM((ROWS, LATENT), jnp.float32),
            pltpu.SemaphoreType.DMA,
        ],
    )
    def k(ids_hbm, table_hbm, out1, out2, out3,
          raw_v, idx1_v, idx2_v, idx3_v, rows1_v, rows2_v, rows3_v, sem):
        wid = lax.axis_index("s") * NC + lax.axis_index("c")
        base = wid * ROWS

        prev = lax.rem(2 * base - PAD + 2 * BATCH, 2 * BATCH)
        pltpu.sync_copy(ids_hbm.at[pl.ds(prev, PAD)], raw_v.at[pl.ds(0, PAD)])
        pltpu.sync_copy(ids_hbm.at[pl.ds(2 * base, 2 * ROWS)],
                        raw_v.at[pl.ds(PAD, 2 * ROWS)])

        lane = lax.iota(jnp.int32, L)
        for t in range(ROWS // L):
            j = 2 * (t * L) + 2 * lane
            v3 = plsc.load_gather(raw_v, [j + (PAD - 1)])
            v1 = plsc.load_gather(raw_v, [j + PAD])
            v2 = plsc.load_gather(raw_v, [j + (PAD + 1)])
            r = (t * L) // KCH
            c = (t * L) % KCH
            idx1_v[r, pl.ds(c, L)] = v1
            idx2_v[r, pl.ds(c, L)] = v2
            idx3_v[r, pl.ds(c, L)] = v3

        copies = []
        for j in range(NCH):
            copies.append(pltpu.make_async_copy(
                table_hbm.at[idx1_v.at[j]],
                rows1_v.at[pl.ds(j * KCH, KCH)], sem))
            copies.append(pltpu.make_async_copy(
                table_hbm.at[idx2_v.at[j]],
                rows2_v.at[pl.ds(j * KCH, KCH)], sem))
            copies.append(pltpu.make_async_copy(
                table_hbm.at[idx3_v.at[j]],
                rows3_v.at[pl.ds(j * KCH, KCH)], sem))
        for cp in copies:
            cp.start()
        for cp in copies:
            cp.wait()

        pltpu.sync_copy(rows1_v, out1.at[pl.ds(base, ROWS)])
        pltpu.sync_copy(rows2_v, out2.at[pl.ds(base, ROWS)])
        pltpu.sync_copy(rows3_v, out3.at[pl.ds(base, ROWS)])

    return k(ids_flat, table)


def kernel(input_ids, table):
    ids_flat = input_ids.astype(jnp.int32).reshape(2 * BATCH)
    return _sc_lookup(ids_flat, table)
